# baseline (speedup 1.0000x reference)
import jax
import jax.numpy as jnp
from jax.experimental import pallas as pl
from jax.experimental.pallas import tpu as pltpu


def _l2norm(x, eps=1e-12):
    d = jnp.sqrt(jnp.sum(x * x, axis=(2, 3), keepdims=True))
    return x / jnp.maximum(d, eps)


def _stats(x):
    rgb, ir = x[0], x[1]
    rgb_cap = jnp.mean(rgb, axis=1, keepdims=True)
    rgb_cmp = jnp.max(rgb, axis=1, keepdims=True)
    ir_cap = jnp.mean(ir, axis=1, keepdims=True)
    ir_cmp = jnp.max(ir, axis=1, keepdims=True)
    x1_cp = jnp.concatenate([rgb_cap, rgb_cmp], axis=1)
    x2_cp = jnp.concatenate([ir_cap, ir_cmp], axis=1)
    cp = x1_cp + x2_cp
    sa = jnp.maximum(cp[:, ::2, :, :], cp[:, 1::2, :, :])
    sa_sig = jax.nn.sigmoid(sa)
    sa_norm = _l2norm(sa_sig)
    sim_rgb = jnp.sum(sa_norm * _l2norm(rgb), axis=(2, 3))
    sim_ir = jnp.sum(sa_norm * _l2norm(ir), axis=(2, 3))
    return sa, sim_rgb, sim_ir


def _kmain(idx_ref, karr_ref, x_ref, out0_ref, out1_ref,
           minrow_ref):
    C = x_ref.shape[2]
    s = pl.program_id(1)
    idxcol = idx_ref[0, 0]
    iota_row = jax.lax.broadcasted_iota(jnp.int32, (1, C), 1)
    iota_col = jax.lax.broadcasted_iota(jnp.int32, (C, 1), 0)
    k0 = karr_ref[0, 0]
    k1 = karr_ref[1, 0]
    act0 = (k1 > k0) & (k0 > 0)
    act1 = (k0 > k1) & (k1 > 0)
    is0 = s == 0
    act = jnp.where(is0, act0, act1)
    kk = jnp.where(is0, k0, k1)

    idx_shift = jnp.concatenate([idxcol[C - 1:], idxcol[:C - 1]], axis=0)
    idx_eff = jnp.where(act & (iota_col > kk), idx_shift, idxcol)

    xb = x_ref[0, 0]
    P = (idx_eff == iota_row).astype(jnp.float32)
    out = jax.lax.dot_general(
        P, xb, (((1,), (0,)), ((), ())),
        preferred_element_type=jnp.float32)

    ohmin = (iota_row == idxcol[0, 0]).astype(jnp.float32)
    rowmin = jax.lax.dot_general(
        ohmin, xb, (((1,), (0,)), ((), ())),
        preferred_element_type=jnp.float32)

    @pl.when(is0)
    def _():
        out0_ref[0] = out
        minrow_ref[...] = rowmin

    @pl.when(jnp.logical_not(is0))
    def _():
        ef = jnp.maximum(minrow_ref[...], rowmin)
        out1_ref[0] = jnp.where(act1 & (iota_col == k1), ef, out)

        @pl.when(act0)
        def _():
            out0_ref[0] = jnp.where(iota_col == k0, ef, out0_ref[0])


def kernel(x):
    S, B, C, H, W = x.shape
    HW = H * W
    f32 = jnp.float32

    sa, sim_rgb, sim_ir = _stats(x)
    sa_sig = jax.nn.sigmoid(sa)
    idx_rgb = jnp.argsort(sim_rgb, axis=1)
    idx_ir = jnp.argsort(sim_ir, axis=1)
    k_rgb = jnp.max(jnp.sum(sim_rgb > 0, axis=1))
    k_ir = jnp.max(jnp.sum(sim_ir > 0, axis=1))

    idxcol = jnp.stack([idx_rgb, idx_ir]).reshape(S, B, C, 1)
    idxcol = idxcol.astype(jnp.int32)
    karr = (jnp.zeros((2, 128), jnp.int32)
            + jnp.stack([k_rgb, k_ir]).reshape(2, 1).astype(jnp.int32))
    xr = x.reshape(S, B, C, HW)

    out0, out1 = pl.pallas_call(
        _kmain,
        grid=(B, S),
        in_specs=[
            pl.BlockSpec((1, 1, C, 1), lambda b, s: (s, b, 0, 0)),
            pl.BlockSpec((2, 128), lambda b, s: (0, 0)),
            pl.BlockSpec((1, 1, C, HW), lambda b, s: (s, b, 0, 0)),
        ],
        out_specs=[pl.BlockSpec((1, C, HW), lambda b, s: (b, 0, 0)),
                   pl.BlockSpec((1, C, HW), lambda b, s: (b, 0, 0))],
        out_shape=[jax.ShapeDtypeStruct((B, C, HW), f32),
                   jax.ShapeDtypeStruct((B, C, HW), f32)],
        scratch_shapes=[pltpu.VMEM((1, HW), f32)],
    )(idxcol, karr, xr)

    return (out0.reshape(B, C, H, W) * sa_sig,
            out1.reshape(B, C, H, W) * sa_sig)
